# baseline (speedup 1.0000x reference)
import functools

import jax
import jax.numpy as jnp
from jax.experimental import pallas as pl
from jax.experimental.pallas import tpu as pltpu

_N_NODES = 100000
_D_FEAT = 128
_N_CLUSTERS = 16
_ALPHA = 10.0
_BLOCK_ROWS = 2000


def _softkmeans_block(x_ref, c_ref, out_ref):
    x = x_ref[...]
    c = c_ref[...]
    xc = jax.lax.dot_general(
        x, c, (((1,), (1,)), ((), ())), preferred_element_type=jnp.float32
    )
    x_sq = jnp.sum(x * x, axis=1, keepdims=True)
    c_sq = jnp.sum(c * c, axis=1)[None, :]
    dist = jnp.maximum(x_sq + c_sq - 2.0 * xc, 0.0)
    m = jnp.min(dist, axis=1, keepdims=True)
    e = jnp.exp(-_ALPHA * (dist - m))
    w = e / jnp.sum(e, axis=1, keepdims=True)
    partial = jnp.sum(w * dist)

    @pl.when(pl.program_id(0) == 0)
    def _():
        out_ref[...] = jnp.zeros_like(out_ref)

    out_ref[...] += partial


@jax.jit
def kernel(x, centers):
    n, d = x.shape
    k = centers.shape[0]
    grid = (n // _BLOCK_ROWS,)
    out = pl.pallas_call(
        _softkmeans_block,
        grid=grid,
        in_specs=[
            pl.BlockSpec((_BLOCK_ROWS, d), lambda i: (i, 0)),
            pl.BlockSpec((k, d), lambda i: (0, 0)),
        ],
        out_specs=pl.BlockSpec((1, 1), lambda i: (0, 0)),
        out_shape=jax.ShapeDtypeStruct((1, 1), jnp.float32),
        compiler_params=pltpu.CompilerParams(
            dimension_semantics=("arbitrary",),
        ),
    )(x, centers)
    return out[0, 0] / n


# speedup vs baseline: 1.5132x; 1.5132x over previous
import jax
import jax.numpy as jnp
from jax.experimental import pallas as pl
from jax.experimental.pallas import tpu as pltpu

_ALPHA = 10.0
_BLOCK_ROWS = 2000


def _softkmeans_block(x_ref, c_ref, out_ref):
    x = x_ref[...]
    c = c_ref[...]
    cx = jax.lax.dot_general(
        c, x, (((1,), (1,)), ((), ())), preferred_element_type=jnp.float32
    )
    c_sq = jnp.sum(c * c, axis=1, keepdims=True)
    t = c_sq - 2.0 * cx
    m = jnp.min(t, axis=0, keepdims=True)
    e = jnp.exp(-_ALPHA * (t - m))
    s = jnp.sum(e, axis=0, keepdims=True)
    wt = jnp.sum(e * t, axis=0, keepdims=True) / s
    partial = jnp.sum(wt) + jnp.sum(x * x)

    @pl.when(pl.program_id(0) == 0)
    def _():
        out_ref[...] = jnp.zeros_like(out_ref)

    out_ref[...] += partial


@jax.jit
def kernel(x, centers):
    n, d = x.shape
    k = centers.shape[0]
    grid = (n // _BLOCK_ROWS,)
    out = pl.pallas_call(
        _softkmeans_block,
        grid=grid,
        in_specs=[
            pl.BlockSpec((_BLOCK_ROWS, d), lambda i: (i, 0)),
            pl.BlockSpec((k, d), lambda i: (0, 0)),
        ],
        out_specs=pl.BlockSpec((1, 1), lambda i: (0, 0)),
        out_shape=jax.ShapeDtypeStruct((1, 1), jnp.float32),
        compiler_params=pltpu.CompilerParams(
            dimension_semantics=("arbitrary",),
        ),
    )(x, centers)
    return out[0, 0] / n


# speedup vs baseline: 2.3658x; 1.5635x over previous
import jax
import jax.numpy as jnp
from jax.experimental import pallas as pl
from jax.experimental.pallas import tpu as pltpu

_ALPHA = 10.0
_BLOCK_ROWS = 5000


def _softkmeans_block(x_ref, c_ref, out_ref):
    x = x_ref[...]
    c = c_ref[...]
    cx = jax.lax.dot_general(
        c, x, (((1,), (1,)), ((), ())), preferred_element_type=jnp.float32
    )
    c_sq = jnp.sum(c * c, axis=1, keepdims=True)
    t = c_sq - 2.0 * cx
    m = jnp.min(t, axis=0, keepdims=True)
    e = jnp.exp(-_ALPHA * (t - m))
    s = jnp.sum(e, axis=0, keepdims=True)
    wt = jnp.sum(e * t, axis=0, keepdims=True) / s
    partial = jnp.sum(wt) + jnp.sum(x * x)

    @pl.when(pl.program_id(0) == 0)
    def _():
        out_ref[...] = jnp.zeros_like(out_ref)

    out_ref[...] += partial


@jax.jit
def kernel(x, centers):
    n, d = x.shape
    k = centers.shape[0]
    grid = (n // _BLOCK_ROWS,)
    out = pl.pallas_call(
        _softkmeans_block,
        grid=grid,
        in_specs=[
            pl.BlockSpec((_BLOCK_ROWS, d), lambda i: (i, 0)),
            pl.BlockSpec((k, d), lambda i: (0, 0)),
        ],
        out_specs=pl.BlockSpec((1, 1), lambda i: (0, 0)),
        out_shape=jax.ShapeDtypeStruct((1, 1), jnp.float32),
        compiler_params=pltpu.CompilerParams(
            dimension_semantics=("arbitrary",),
        ),
    )(x, centers)
    return out[0, 0] / n


# speedup vs baseline: 2.9106x; 1.2303x over previous
import jax
import jax.numpy as jnp
from jax.experimental import pallas as pl
from jax.experimental.pallas import tpu as pltpu

_ALPHA = 10.0
_BLOCK_ROWS = 10000


def _softkmeans_block(x_ref, c_ref, out_ref):
    x = x_ref[...]
    c = c_ref[...]
    cx = jax.lax.dot_general(
        c, x, (((1,), (1,)), ((), ())), preferred_element_type=jnp.float32
    )
    c_sq = jnp.sum(c * c, axis=1, keepdims=True)
    t = c_sq - 2.0 * cx
    m = jnp.min(t, axis=0, keepdims=True)
    e = jnp.exp(-_ALPHA * (t - m))
    s = jnp.sum(e, axis=0, keepdims=True)
    wt = jnp.sum(e * t, axis=0, keepdims=True) / s
    partial = jnp.sum(wt) + jnp.sum(x * x)

    @pl.when(pl.program_id(0) == 0)
    def _():
        out_ref[...] = jnp.zeros_like(out_ref)

    out_ref[...] += partial


@jax.jit
def kernel(x, centers):
    n, d = x.shape
    k = centers.shape[0]
    grid = (n // _BLOCK_ROWS,)
    out = pl.pallas_call(
        _softkmeans_block,
        grid=grid,
        in_specs=[
            pl.BlockSpec((_BLOCK_ROWS, d), lambda i: (i, 0)),
            pl.BlockSpec((k, d), lambda i: (0, 0)),
        ],
        out_specs=pl.BlockSpec((1, 1), lambda i: (0, 0)),
        out_shape=jax.ShapeDtypeStruct((1, 1), jnp.float32),
        compiler_params=pltpu.CompilerParams(
            dimension_semantics=("arbitrary",),
        ),
    )(x, centers)
    return out[0, 0] / n


# speedup vs baseline: 3.0646x; 1.0529x over previous
import jax
import jax.numpy as jnp
from jax.experimental import pallas as pl
from jax.experimental.pallas import tpu as pltpu

_ALPHA = 10.0
_BLOCK_ROWS = 20000


def _softkmeans_block(x_ref, c_ref, out_ref):
    x = x_ref[...]
    c = c_ref[...]
    cx = jax.lax.dot_general(
        c, x, (((1,), (1,)), ((), ())), preferred_element_type=jnp.float32
    )
    c_sq = jnp.sum(c * c, axis=1, keepdims=True)
    t = c_sq - 2.0 * cx
    m = jnp.min(t, axis=0, keepdims=True)
    e = jnp.exp(-_ALPHA * (t - m))
    s = jnp.sum(e, axis=0, keepdims=True)
    wt = jnp.sum(e * t, axis=0, keepdims=True) / s
    partial = jnp.sum(wt) + jnp.sum(x * x)

    @pl.when(pl.program_id(0) == 0)
    def _():
        out_ref[...] = jnp.zeros_like(out_ref)

    out_ref[...] += partial


@jax.jit
def kernel(x, centers):
    n, d = x.shape
    k = centers.shape[0]
    grid = (n // _BLOCK_ROWS,)
    out = pl.pallas_call(
        _softkmeans_block,
        grid=grid,
        in_specs=[
            pl.BlockSpec((_BLOCK_ROWS, d), lambda i: (i, 0)),
            pl.BlockSpec((k, d), lambda i: (0, 0)),
        ],
        out_specs=pl.BlockSpec((1, 1), lambda i: (0, 0)),
        out_shape=jax.ShapeDtypeStruct((1, 1), jnp.float32),
        compiler_params=pltpu.CompilerParams(
            dimension_semantics=("arbitrary",),
        ),
    )(x, centers)
    return out[0, 0] / n


# speedup vs baseline: 3.1116x; 1.0153x over previous
import jax
import jax.numpy as jnp
from jax.experimental import pallas as pl
from jax.experimental.pallas import tpu as pltpu

_ALPHA = 10.0
_BLOCK_ROWS = 25000


def _softkmeans_block(x_ref, c_ref, out_ref):
    x = x_ref[...]
    c = c_ref[...]
    cx = jax.lax.dot_general(
        c, x, (((1,), (1,)), ((), ())), preferred_element_type=jnp.float32
    )
    c_sq = jnp.sum(c * c, axis=1, keepdims=True)
    t = c_sq - 2.0 * cx
    m = jnp.min(t, axis=0, keepdims=True)
    e = jnp.exp(-_ALPHA * (t - m))
    s = jnp.sum(e, axis=0, keepdims=True)
    wt = jnp.sum(e * t, axis=0, keepdims=True) / s
    partial = jnp.sum(wt) + jnp.sum(x * x)

    @pl.when(pl.program_id(0) == 0)
    def _():
        out_ref[...] = jnp.zeros_like(out_ref)

    out_ref[...] += partial


@jax.jit
def kernel(x, centers):
    n, d = x.shape
    k = centers.shape[0]
    grid = (n // _BLOCK_ROWS,)
    out = pl.pallas_call(
        _softkmeans_block,
        grid=grid,
        in_specs=[
            pl.BlockSpec((_BLOCK_ROWS, d), lambda i: (i, 0)),
            pl.BlockSpec((k, d), lambda i: (0, 0)),
        ],
        out_specs=pl.BlockSpec((1, 1), lambda i: (0, 0)),
        out_shape=jax.ShapeDtypeStruct((1, 1), jnp.float32),
        compiler_params=pltpu.CompilerParams(
            dimension_semantics=("arbitrary",),
        ),
    )(x, centers)
    return out[0, 0] / n
